# baseline (speedup 1.0000x reference)
import functools

import jax
import jax.numpy as jnp
from jax import lax
from jax.experimental import pallas as pl
from jax.experimental.pallas import tpu as pltpu
from jax.experimental.pallas import tpu_sc as plsc

NC = 2
NS = 16
CB = 128
GRP = 8


def _w2v_body(tgt_hbm, ctx_hbm, ttab, ctab, out_hbm,
              tidx_v, cidx_v, te_v, ce_v, out_v, sem,
              *, b_per_w, c_dim, e_dim):
    wid = lax.axis_index("s") * NC + lax.axis_index("c")
    n_chunks = b_per_w // CB
    nvec = (GRP * c_dim) // 16
    lanes = lax.iota(jnp.int32, 16)
    for chunk in range(n_chunks):
        base = wid * b_per_w + chunk * CB
        pltpu.sync_copy(tgt_hbm.at[pl.ds(base, CB)], tidx_v)
        pltpu.sync_copy(ctx_hbm.at[pl.ds(base * c_dim, CB * c_dim)], cidx_v)
        copies = [pltpu.async_copy(ttab.at[tidx_v], te_v, sem)]
        for j in range(c_dim):
            copies.append(
                pltpu.async_copy(ctab.at[cidx_v.at[pl.ds(j * CB, CB)]],
                                 ce_v.at[pl.ds(j * CB, CB)], sem))
        for cp in copies:
            cp.wait()

        for blk in range(CB // 16):
            rows16 = blk * 16 + lanes
            crow = [rows16 * c_dim + c for c in range(c_dim)]
            zero = jnp.zeros((16,), jnp.float32)

            def ebody(e, accs):
                ecol = jnp.full((16,), e, jnp.int32)
                tg = plsc.load_gather(te_v, [rows16, ecol])
                return tuple(
                    accs[c] + tg * plsc.load_gather(ce_v, [crow[c], ecol])
                    for c in range(c_dim))

            accs = lax.fori_loop(0, e_dim, ebody, (zero,) * c_dim)
            for c in range(c_dim):
                plsc.store_scatter(out_v, [crow[c]], accs[c])
        pltpu.sync_copy(out_v, out_hbm.at[pl.ds(base * c_dim, CB * c_dim)])


def kernel(target, context, target_table, context_table):
    b_dim = target.shape[0]
    c_dim = context.shape[1]
    e_dim = target_table.shape[1]
    nw = NC * NS
    b_per_w = b_dim // nw

    ctx_flat = context.reshape(b_dim * c_dim)

    mesh = plsc.VectorSubcoreMesh(core_axis_name="c", subcore_axis_name="s")
    run = functools.partial(
        pl.kernel,
        mesh=mesh,
        compiler_params=pltpu.CompilerParams(needs_layout_passes=False,
                                             use_tc_tiling_on_sc=False),
        out_type=jax.ShapeDtypeStruct((b_dim * c_dim,), jnp.float32),
        scratch_types=[
            pltpu.VMEM((CB,), jnp.int32),
            pltpu.VMEM((CB * c_dim,), jnp.int32),
            pltpu.VMEM((CB, e_dim), jnp.float32),
            pltpu.VMEM((CB * c_dim, e_dim), jnp.float32),
            pltpu.VMEM((CB * c_dim,), jnp.float32),
            pltpu.SemaphoreType.DMA,
        ],
    )(functools.partial(_w2v_body, b_per_w=b_per_w, c_dim=c_dim, e_dim=e_dim))
    out = run(target, ctx_flat, target_table, context_table)
    return out.reshape(b_dim, c_dim)
